# baseline (speedup 1.0000x reference)
import functools

import jax
import jax.numpy as jnp
from jax import lax
from jax.experimental import pallas as pl
from jax.experimental.pallas import tpu as pltpu
from jax.experimental.pallas import tpu_sc as plsc

N = 10000
E = 320000
NC = 2
NS = 16
NW = NC * NS
KB = 128
NBLK = E // KB

N_PAD = 10240
ZROWS = 128

DEG_NB = NBLK // NW
DEG_XTRA = NBLK - DEG_NB * NW

FS_NB = NBLK // NS
FS_XTRA = NBLK - FS_NB * NS
FS_PH = 52

_MESH = plsc.VectorSubcoreMesh(
    core_axis_name="c", subcore_axis_name="s", num_cores=NC, num_subcores=NS)

_SC_PARAMS = pltpu.CompilerParams(use_tc_tiling_on_sc=False)


def _worker_id():
  c = lax.axis_index("c")
  s = lax.axis_index("s")
  return c * NS + s, c, s


@functools.partial(
    pl.kernel,
    out_type=jax.ShapeDtypeStruct((NC, N_PAD), jnp.float32),
    mesh=_MESH,
    compiler_params=_SC_PARAMS,
    scratch_types=[
        pltpu.VMEM((DEG_NB, KB), jnp.int32),
        pltpu.VMEM((KB,), jnp.float32),
        pltpu.VMEM((640,), jnp.float32),
        pltpu.VMEM_SHARED((N_PAD,), jnp.float32),
        pltpu.SemaphoreType.DMA,
    ],
)
def _sc_degree(ei3_hbm, out_hbm, id2_v, ones_v, zb_v, deg_sh, sem):
  wid, c, s = _worker_id()

  for i in range(KB // 16):
    ones_v[pl.ds(i * 16, 16)] = jnp.full((16,), 1.0, jnp.float32)
  for i in range(640 // 16):
    zb_v[pl.ds(i * 16, 16)] = jnp.zeros((16,), jnp.float32)

  pltpu.sync_copy(zb_v, deg_sh.at[pl.ds(s * 640, 640)])
  pltpu.sync_copy(ei3_hbm.at[1, pl.ds(wid * DEG_NB, DEG_NB)], id2_v)
  plsc.subcore_barrier()

  def body(g, carry):
    hs = [pltpu.async_copy(ones_v, deg_sh.at[id2_v.at[g * 6 + b]], sem,
                           add=True) for b in range(6)]
    for h in hs:
      h.wait()
    return carry

  lax.fori_loop(0, DEG_NB // 6, body, 0)

  @pl.when(wid < DEG_XTRA)
  def _():
    pltpu.sync_copy(ei3_hbm.at[1, pl.ds(NW * DEG_NB + wid, 1)],
                    id2_v.at[pl.ds(0, 1)])
    pltpu.sync_copy(ones_v, deg_sh.at[id2_v.at[0]], add=True)

  plsc.subcore_barrier()

  pltpu.sync_copy(deg_sh.at[pl.ds(s * 640, 640)], zb_v)
  pltpu.sync_copy(zb_v, out_hbm.at[c, pl.ds(s * 640, 640)])


HBM_TILES = 5


def _make_sc_prop_fs(d):
  dh = d // 2
  half = jax.ShapeDtypeStruct((N_PAD, dh), jnp.float32)

  @functools.partial(
      pl.kernel,
      out_type=[half, half],
      mesh=_MESH,
      compiler_params=_SC_PARAMS,
      scratch_types=[
          pltpu.VMEM((FS_PH, KB), jnp.int32),
          pltpu.VMEM((FS_PH, KB), jnp.int32),
          pltpu.VMEM((KB, dh), jnp.float32),
          pltpu.VMEM((KB, dh), jnp.float32),
          pltpu.VMEM_SHARED((N_PAD, dh), jnp.float32),
          pltpu.VMEM_SHARED((N_PAD, dh), jnp.float32),
          pltpu.SemaphoreType.DMA,
          pltpu.SemaphoreType.DMA,
      ],
  )
  def prop(g3_hbm, ei3_hbm, out0_hbm, out1_hbm, is2_v, id2_v, rows0_v,
           rows1_v, g_sh, acc_sh, semg0, semg1):
    _, c, s = _worker_id()
    g_my = g3_hbm.at[c]

    offs = list(range(0, dh - 15, 16))
    if offs[-1] != dh - 16:
      offs.append(dh - 16)
    def zinit(r, carry):
      for col in offs:
        rows0_v[r, pl.ds(col, 16)] = jnp.zeros((16,), jnp.float32)
      return carry
    lax.fori_loop(0, KB, zinit, 0)
    for j in range(5):
      pltpu.sync_copy(rows0_v, acc_sh.at[pl.ds(s * 640 + j * ZROWS, ZROWS)])
    for j in range(5):
      r0 = s * 640 + j * ZROWS
      pltpu.sync_copy(g_my.at[pl.ds(r0, ZROWS)], rows1_v)
      pltpu.sync_copy(rows1_v, g_sh.at[pl.ds(r0, ZROWS)])
    plsc.subcore_barrier()

    def run_phases(gsrc):
      for ph in range(FS_NB // FS_PH):
        base = s * FS_NB + ph * FS_PH
        pltpu.sync_copy(ei3_hbm.at[0, pl.ds(base, FS_PH)], is2_v)
        pltpu.sync_copy(ei3_hbm.at[1, pl.ds(base, FS_PH)], id2_v)
        pltpu.async_copy(gsrc.at[is2_v.at[0]], rows0_v, semg0)

        def body(p, carry):
          j0 = p * 2
          pltpu.async_copy(gsrc.at[is2_v.at[j0 + 1]], rows1_v, semg1)
          pltpu.make_async_copy(gsrc.at[is2_v.at[j0]], rows0_v, semg0).wait()
          pltpu.sync_copy(rows0_v, acc_sh.at[id2_v.at[j0]], add=True)

          @pl.when(p < FS_PH // 2 - 1)
          def _():
            pltpu.async_copy(gsrc.at[is2_v.at[j0 + 2]], rows0_v, semg0)

          pltpu.make_async_copy(gsrc.at[is2_v.at[j0 + 1]], rows1_v,
                                semg1).wait()
          pltpu.sync_copy(rows1_v, acc_sh.at[id2_v.at[j0 + 1]], add=True)
          return carry

        lax.fori_loop(0, FS_PH // 2, body, 0)

    @pl.when(s < HBM_TILES)
    def _():
      run_phases(g_my)

    @pl.when(s >= HBM_TILES)
    def _():
      run_phases(g_sh)

    @pl.when(s < FS_XTRA)
    def _():
      blk = NS * FS_NB + s
      pltpu.sync_copy(ei3_hbm.at[0, pl.ds(blk, 1)], is2_v.at[pl.ds(0, 1)])
      pltpu.sync_copy(ei3_hbm.at[1, pl.ds(blk, 1)], id2_v.at[pl.ds(0, 1)])
      pltpu.async_copy(g_sh.at[is2_v.at[0]], rows0_v, semg0).wait()
      pltpu.sync_copy(rows0_v, acc_sh.at[id2_v.at[0]], add=True)

    plsc.subcore_barrier()

    for j in range(5):
      r0 = s * 640 + j * ZROWS
      pltpu.sync_copy(acc_sh.at[pl.ds(r0, ZROWS)], rows0_v)

      @pl.when(c == 0)
      def _():
        pltpu.sync_copy(rows0_v, out0_hbm.at[pl.ds(r0, ZROWS)])

      @pl.when(c == 1)
      def _():
        pltpu.sync_copy(rows0_v, out1_hbm.at[pl.ds(r0, ZROWS)])

  return prop


_sc_prop_fs_128 = _make_sc_prop_fs(128)
_sc_prop_fs_48 = _make_sc_prop_fs(48)


_RB = 1280


def _tc1a_body(x_ref, w1_ref, h1_ref):
  h1_ref[...] = lax.dot_general(x_ref[...], w1_ref[...],
                                (((1,), (1,)), ((), ())),
                                preferred_element_type=jnp.float32)


def _tc1a(x, w1):
  return pl.pallas_call(
      _tc1a_body,
      grid=(N_PAD // _RB,),
      in_specs=[
          pl.BlockSpec((_RB, 128), lambda i: (i, 0)),
          pl.BlockSpec((128, 128), lambda i: (0, 0)),
      ],
      out_specs=pl.BlockSpec((_RB, 128), lambda i: (i, 0)),
      out_shape=jax.ShapeDtypeStruct((N_PAD, 128), jnp.float32),
  )(x, w1)


def _tc1b_body(h1_ref, d0_ref, d1_ref, g3_ref, dinv_ref):
  deg = d0_ref[...] + d1_ref[...] + 1.0
  dinv = lax.rsqrt(jnp.maximum(deg, 1.0))
  g = dinv * h1_ref[...]
  g3_ref[0] = g[:, :64]
  g3_ref[1] = g[:, 64:]
  dinv_ref[...] = dinv


def _tc1b(h1, d0, d1):
  return pl.pallas_call(
      _tc1b_body,
      grid=(N_PAD // _RB,),
      in_specs=[
          pl.BlockSpec((_RB, 128), lambda i: (i, 0)),
          pl.BlockSpec((_RB, 1), lambda i: (i, 0)),
          pl.BlockSpec((_RB, 1), lambda i: (i, 0)),
      ],
      out_specs=[
          pl.BlockSpec((2, _RB, 64), lambda i: (0, i, 0)),
          pl.BlockSpec((_RB, 1), lambda i: (i, 0)),
      ],
      out_shape=[
          jax.ShapeDtypeStruct((2, N_PAD, 64), jnp.float32),
          jax.ShapeDtypeStruct((N_PAD, 1), jnp.float32),
      ],
  )(h1, d0, d1)


def _tc2_body(h0_ref, h1_ref, g3_ref, dinv_ref, b1_ref, w2_ref, g4_ref):
  dinv = dinv_ref[...]
  s1 = jnp.concatenate([h0_ref[...], h1_ref[...]], axis=1)
  g1 = jnp.concatenate([g3_ref[0], g3_ref[1]], axis=1)
  t = (s1 + g1) * dinv + b1_ref[...]
  o1 = jnp.maximum(t, 0.0)
  h2 = lax.dot_general(o1, w2_ref[...], (((1,), (1,)), ((), ())),
                       preferred_element_type=jnp.float32)
  g2 = dinv * h2
  g4_ref[0] = g2[:, :24]
  g4_ref[1] = g2[:, 24:]


def _tc2(h0, h1, g3, dinv, b1, w2p):
  return pl.pallas_call(
      _tc2_body,
      grid=(N_PAD // _RB,),
      in_specs=[
          pl.BlockSpec((_RB, 64), lambda i: (i, 0)),
          pl.BlockSpec((_RB, 64), lambda i: (i, 0)),
          pl.BlockSpec((2, _RB, 64), lambda i: (0, i, 0)),
          pl.BlockSpec((_RB, 1), lambda i: (i, 0)),
          pl.BlockSpec((1, 128), lambda i: (0, 0)),
          pl.BlockSpec((48, 128), lambda i: (0, 0)),
      ],
      out_specs=pl.BlockSpec((2, _RB, 24), lambda i: (0, i, 0)),
      out_shape=jax.ShapeDtypeStruct((2, N_PAD, 24), jnp.float32),
  )(h0, h1, g3, dinv, b1, w2p)


def _tc3_body(h0_ref, h1_ref, g4_ref, dinv_ref, b2_ref, out_ref):
  s2 = jnp.concatenate([h0_ref[...], h1_ref[...]], axis=1)
  g2 = jnp.concatenate([g4_ref[0], g4_ref[1]], axis=1)
  out_ref[...] = (s2 + g2) * dinv_ref[...] + b2_ref[...]


def _tc3(h0, h1, g4, dinv, b2p):
  return pl.pallas_call(
      _tc3_body,
      grid=(N_PAD // _RB,),
      in_specs=[
          pl.BlockSpec((_RB, 24), lambda i: (i, 0)),
          pl.BlockSpec((_RB, 24), lambda i: (i, 0)),
          pl.BlockSpec((2, _RB, 24), lambda i: (0, i, 0)),
          pl.BlockSpec((_RB, 1), lambda i: (i, 0)),
          pl.BlockSpec((1, 48), lambda i: (0, 0)),
      ],
      out_specs=pl.BlockSpec((_RB, 48), lambda i: (i, 0)),
      out_shape=jax.ShapeDtypeStruct((N_PAD, 48), jnp.float32),
  )(h0, h1, g4, dinv, b2p)


def kernel(x, edge_index, W1, b1, W2, b2):
  ei3 = edge_index.astype(jnp.int32).reshape(2, NBLK, KB)

  h1 = _tc1a(x, W1)
  degp = _sc_degree(ei3)
  g3, dinv = _tc1b(h1, degp[0][:, None], degp[1][:, None])

  s1h0, s1h1 = _sc_prop_fs_128(g3, ei3)

  w2p = jnp.zeros((48, 128), jnp.float32).at[:40].set(W2)
  b1r = b1.reshape(1, 128)
  b2p = jnp.zeros((1, 48), jnp.float32).at[0, :40].set(b2)

  g4 = _tc2(s1h0, s1h1, g3, dinv, b1r, w2p)
  s2h0, s2h1 = _sc_prop_fs_48(g4, ei3)
  out = _tc3(s2h0, s2h1, g4, dinv, b2p)
  return out[:N, :40]


# speedup vs baseline: 1.1475x; 1.1475x over previous
import functools

import jax
import jax.numpy as jnp
from jax import lax
from jax.experimental import pallas as pl
from jax.experimental.pallas import tpu as pltpu
from jax.experimental.pallas import tpu_sc as plsc

N = 10000
E = 320000
NC = 2
NS = 16
NW = NC * NS
KB = 128
NBLK = E // KB

N_PAD = 10240
ZROWS = 128

DEG_NB = NBLK // NW
DEG_XTRA = NBLK - DEG_NB * NW

FS_NB = NBLK // NS
FS_XTRA = NBLK - FS_NB * NS
FS_PH = 52

_MESH = plsc.VectorSubcoreMesh(
    core_axis_name="c", subcore_axis_name="s", num_cores=NC, num_subcores=NS)

_SC_PARAMS = pltpu.CompilerParams(use_tc_tiling_on_sc=False)


def _worker_id():
  c = lax.axis_index("c")
  s = lax.axis_index("s")
  return c * NS + s, c, s


@functools.partial(
    pl.kernel,
    out_type=jax.ShapeDtypeStruct((NC, N_PAD), jnp.float32),
    mesh=_MESH,
    compiler_params=_SC_PARAMS,
    scratch_types=[
        pltpu.VMEM((DEG_NB, KB), jnp.int32),
        pltpu.VMEM((KB,), jnp.float32),
        pltpu.VMEM((640,), jnp.float32),
        pltpu.VMEM_SHARED((N_PAD,), jnp.float32),
        pltpu.SemaphoreType.DMA,
    ],
)
def _sc_degree(ei3_hbm, out_hbm, id2_v, ones_v, zb_v, deg_sh, sem):
  wid, c, s = _worker_id()

  for i in range(KB // 16):
    ones_v[pl.ds(i * 16, 16)] = jnp.full((16,), 1.0, jnp.float32)
  for i in range(640 // 16):
    zb_v[pl.ds(i * 16, 16)] = jnp.zeros((16,), jnp.float32)

  pltpu.sync_copy(zb_v, deg_sh.at[pl.ds(s * 640, 640)])
  pltpu.sync_copy(ei3_hbm.at[1, pl.ds(wid * DEG_NB, DEG_NB)], id2_v)
  plsc.subcore_barrier()

  def body(g, carry):
    hs = [pltpu.async_copy(ones_v, deg_sh.at[id2_v.at[g * 6 + b]], sem,
                           add=True) for b in range(6)]
    for h in hs:
      h.wait()
    return carry

  lax.fori_loop(0, DEG_NB // 6, body, 0)

  @pl.when(wid < DEG_XTRA)
  def _():
    pltpu.sync_copy(ei3_hbm.at[1, pl.ds(NW * DEG_NB + wid, 1)],
                    id2_v.at[pl.ds(0, 1)])
    pltpu.sync_copy(ones_v, deg_sh.at[id2_v.at[0]], add=True)

  plsc.subcore_barrier()

  pltpu.sync_copy(deg_sh.at[pl.ds(s * 640, 640)], zb_v)
  pltpu.sync_copy(zb_v, out_hbm.at[c, pl.ds(s * 640, 640)])


HBM_TILES = 0


def _make_sc_prop_fs(d):
  dh = d // 2
  half = jax.ShapeDtypeStruct((N_PAD, dh), jnp.float32)

  @functools.partial(
      pl.kernel,
      out_type=[half, half],
      mesh=_MESH,
      compiler_params=_SC_PARAMS,
      scratch_types=[
          pltpu.VMEM((FS_PH, KB), jnp.int32),
          pltpu.VMEM((FS_PH, KB), jnp.int32),
          pltpu.VMEM((KB, dh), jnp.float32),
          pltpu.VMEM((KB, dh), jnp.float32),
          pltpu.VMEM_SHARED((N_PAD, dh), jnp.float32),
          pltpu.VMEM_SHARED((N_PAD, dh), jnp.float32),
          pltpu.SemaphoreType.DMA,
          pltpu.SemaphoreType.DMA,
      ],
  )
  def prop(g3_hbm, ei3_hbm, out0_hbm, out1_hbm, is2_v, id2_v, rows0_v,
           rows1_v, g_sh, acc_sh, semg0, semg1):
    _, c, s = _worker_id()
    g_my = g3_hbm.at[c]

    offs = list(range(0, dh - 15, 16))
    if offs[-1] != dh - 16:
      offs.append(dh - 16)
    def zinit(r, carry):
      for col in offs:
        rows0_v[r, pl.ds(col, 16)] = jnp.zeros((16,), jnp.float32)
      return carry
    lax.fori_loop(0, KB, zinit, 0)
    for j in range(5):
      pltpu.sync_copy(rows0_v, acc_sh.at[pl.ds(s * 640 + j * ZROWS, ZROWS)])
    for j in range(5):
      r0 = s * 640 + j * ZROWS
      pltpu.sync_copy(g_my.at[pl.ds(r0, ZROWS)], rows1_v)
      pltpu.sync_copy(rows1_v, g_sh.at[pl.ds(r0, ZROWS)])
    plsc.subcore_barrier()

    def run_phases(gsrc):
      for ph in range(FS_NB // FS_PH):
        base = s * FS_NB + ph * FS_PH
        pltpu.sync_copy(ei3_hbm.at[0, pl.ds(base, FS_PH)], is2_v)
        pltpu.sync_copy(ei3_hbm.at[1, pl.ds(base, FS_PH)], id2_v)
        pltpu.async_copy(gsrc.at[is2_v.at[0]], rows0_v, semg0)

        def body(p, carry):
          j0 = p * 2
          pltpu.async_copy(gsrc.at[is2_v.at[j0 + 1]], rows1_v, semg1)
          pltpu.make_async_copy(gsrc.at[is2_v.at[j0]], rows0_v, semg0).wait()
          pltpu.sync_copy(rows0_v, acc_sh.at[id2_v.at[j0]], add=True)

          @pl.when(p < FS_PH // 2 - 1)
          def _():
            pltpu.async_copy(gsrc.at[is2_v.at[j0 + 2]], rows0_v, semg0)

          pltpu.make_async_copy(gsrc.at[is2_v.at[j0 + 1]], rows1_v,
                                semg1).wait()
          pltpu.sync_copy(rows1_v, acc_sh.at[id2_v.at[j0 + 1]], add=True)
          return carry

        lax.fori_loop(0, FS_PH // 2, body, 0)

    @pl.when(s < HBM_TILES)
    def _():
      run_phases(g_my)

    @pl.when(s >= HBM_TILES)
    def _():
      run_phases(g_sh)

    @pl.when(s < FS_XTRA)
    def _():
      blk = NS * FS_NB + s
      pltpu.sync_copy(ei3_hbm.at[0, pl.ds(blk, 1)], is2_v.at[pl.ds(0, 1)])
      pltpu.sync_copy(ei3_hbm.at[1, pl.ds(blk, 1)], id2_v.at[pl.ds(0, 1)])
      pltpu.async_copy(g_sh.at[is2_v.at[0]], rows0_v, semg0).wait()
      pltpu.sync_copy(rows0_v, acc_sh.at[id2_v.at[0]], add=True)

    plsc.subcore_barrier()

    for j in range(5):
      r0 = s * 640 + j * ZROWS
      pltpu.sync_copy(acc_sh.at[pl.ds(r0, ZROWS)], rows0_v)

      @pl.when(c == 0)
      def _():
        pltpu.sync_copy(rows0_v, out0_hbm.at[pl.ds(r0, ZROWS)])

      @pl.when(c == 1)
      def _():
        pltpu.sync_copy(rows0_v, out1_hbm.at[pl.ds(r0, ZROWS)])

  return prop


_sc_prop_fs_128 = _make_sc_prop_fs(128)
_sc_prop_fs_48 = _make_sc_prop_fs(48)


_RB = 1280


def _tc1a_body(x_ref, w1_ref, h1_ref):
  h1_ref[...] = lax.dot_general(x_ref[...], w1_ref[...],
                                (((1,), (1,)), ((), ())),
                                preferred_element_type=jnp.float32)


def _tc1a(x, w1):
  return pl.pallas_call(
      _tc1a_body,
      grid=(N_PAD // _RB,),
      in_specs=[
          pl.BlockSpec((_RB, 128), lambda i: (i, 0)),
          pl.BlockSpec((128, 128), lambda i: (0, 0)),
      ],
      out_specs=pl.BlockSpec((_RB, 128), lambda i: (i, 0)),
      out_shape=jax.ShapeDtypeStruct((N_PAD, 128), jnp.float32),
  )(x, w1)


def _tc1b_body(h1_ref, d0_ref, d1_ref, g3_ref, dinv_ref):
  deg = d0_ref[...] + d1_ref[...] + 1.0
  dinv = lax.rsqrt(jnp.maximum(deg, 1.0))
  g = dinv * h1_ref[...]
  g3_ref[0] = g[:, :64]
  g3_ref[1] = g[:, 64:]
  dinv_ref[...] = dinv


def _tc1b(h1, d0, d1):
  return pl.pallas_call(
      _tc1b_body,
      grid=(N_PAD // _RB,),
      in_specs=[
          pl.BlockSpec((_RB, 128), lambda i: (i, 0)),
          pl.BlockSpec((_RB, 1), lambda i: (i, 0)),
          pl.BlockSpec((_RB, 1), lambda i: (i, 0)),
      ],
      out_specs=[
          pl.BlockSpec((2, _RB, 64), lambda i: (0, i, 0)),
          pl.BlockSpec((_RB, 1), lambda i: (i, 0)),
      ],
      out_shape=[
          jax.ShapeDtypeStruct((2, N_PAD, 64), jnp.float32),
          jax.ShapeDtypeStruct((N_PAD, 1), jnp.float32),
      ],
  )(h1, d0, d1)


def _tc2_body(h0_ref, h1_ref, g3_ref, dinv_ref, b1_ref, w2_ref, g4_ref):
  dinv = dinv_ref[...]
  s1 = jnp.concatenate([h0_ref[...], h1_ref[...]], axis=1)
  g1 = jnp.concatenate([g3_ref[0], g3_ref[1]], axis=1)
  t = (s1 + g1) * dinv + b1_ref[...]
  o1 = jnp.maximum(t, 0.0)
  h2 = lax.dot_general(o1, w2_ref[...], (((1,), (1,)), ((), ())),
                       preferred_element_type=jnp.float32)
  g2 = dinv * h2
  g4_ref[0] = g2[:, :24]
  g4_ref[1] = g2[:, 24:]


def _tc2(h0, h1, g3, dinv, b1, w2p):
  return pl.pallas_call(
      _tc2_body,
      grid=(N_PAD // _RB,),
      in_specs=[
          pl.BlockSpec((_RB, 64), lambda i: (i, 0)),
          pl.BlockSpec((_RB, 64), lambda i: (i, 0)),
          pl.BlockSpec((2, _RB, 64), lambda i: (0, i, 0)),
          pl.BlockSpec((_RB, 1), lambda i: (i, 0)),
          pl.BlockSpec((1, 128), lambda i: (0, 0)),
          pl.BlockSpec((48, 128), lambda i: (0, 0)),
      ],
      out_specs=pl.BlockSpec((2, _RB, 24), lambda i: (0, i, 0)),
      out_shape=jax.ShapeDtypeStruct((2, N_PAD, 24), jnp.float32),
  )(h0, h1, g3, dinv, b1, w2p)


def _tc3_body(h0_ref, h1_ref, g4_ref, dinv_ref, b2_ref, out_ref):
  s2 = jnp.concatenate([h0_ref[...], h1_ref[...]], axis=1)
  g2 = jnp.concatenate([g4_ref[0], g4_ref[1]], axis=1)
  out_ref[...] = (s2 + g2) * dinv_ref[...] + b2_ref[...]


def _tc3(h0, h1, g4, dinv, b2p):
  return pl.pallas_call(
      _tc3_body,
      grid=(N_PAD // _RB,),
      in_specs=[
          pl.BlockSpec((_RB, 24), lambda i: (i, 0)),
          pl.BlockSpec((_RB, 24), lambda i: (i, 0)),
          pl.BlockSpec((2, _RB, 24), lambda i: (0, i, 0)),
          pl.BlockSpec((_RB, 1), lambda i: (i, 0)),
          pl.BlockSpec((1, 48), lambda i: (0, 0)),
      ],
      out_specs=pl.BlockSpec((_RB, 48), lambda i: (i, 0)),
      out_shape=jax.ShapeDtypeStruct((N_PAD, 48), jnp.float32),
  )(h0, h1, g4, dinv, b2p)


def kernel(x, edge_index, W1, b1, W2, b2):
  ei3 = edge_index.astype(jnp.int32).reshape(2, NBLK, KB)

  h1 = _tc1a(x, W1)
  degp = _sc_degree(ei3)
  g3, dinv = _tc1b(h1, degp[0][:, None], degp[1][:, None])

  s1h0, s1h1 = _sc_prop_fs_128(g3, ei3)

  w2p = jnp.zeros((48, 128), jnp.float32).at[:40].set(W2)
  b1r = b1.reshape(1, 128)
  b2p = jnp.zeros((1, 48), jnp.float32).at[0, :40].set(b2)

  g4 = _tc2(s1h0, s1h1, g3, dinv, b1r, w2p)
  s2h0, s2h1 = _sc_prop_fs_48(g4, ei3)
  out = _tc3(s2h0, s2h1, g4, dinv, b2p)
  return out[:N, :40]


# speedup vs baseline: 1.2215x; 1.0645x over previous
import functools

import jax
import jax.numpy as jnp
from jax import lax
from jax.experimental import pallas as pl
from jax.experimental.pallas import tpu as pltpu
from jax.experimental.pallas import tpu_sc as plsc

N = 10000
E = 320000
NC = 2
NS = 16
NW = NC * NS
KB = 128
NBLK = E // KB

N_PAD = 10240
ZROWS = 128

DEG_NB = NBLK // NW
DEG_XTRA = NBLK - DEG_NB * NW

FS_NB = NBLK // NS
FS_XTRA = NBLK - FS_NB * NS
FS_PH = 52

_MESH = plsc.VectorSubcoreMesh(
    core_axis_name="c", subcore_axis_name="s", num_cores=NC, num_subcores=NS)

_SC_PARAMS = pltpu.CompilerParams(use_tc_tiling_on_sc=False)


def _worker_id():
  c = lax.axis_index("c")
  s = lax.axis_index("s")
  return c * NS + s, c, s


@functools.partial(
    pl.kernel,
    out_type=jax.ShapeDtypeStruct((NC, N_PAD), jnp.float32),
    mesh=_MESH,
    compiler_params=_SC_PARAMS,
    scratch_types=[
        pltpu.VMEM((DEG_NB, KB), jnp.int32),
        pltpu.VMEM((KB,), jnp.float32),
        pltpu.VMEM((640,), jnp.float32),
        pltpu.VMEM_SHARED((N_PAD,), jnp.float32),
        pltpu.SemaphoreType.DMA,
    ],
)
def _sc_degree(ei3_hbm, out_hbm, id2_v, ones_v, zb_v, deg_sh, sem):
  wid, c, s = _worker_id()

  for i in range(KB // 16):
    ones_v[pl.ds(i * 16, 16)] = jnp.full((16,), 1.0, jnp.float32)
  for i in range(640 // 16):
    zb_v[pl.ds(i * 16, 16)] = jnp.zeros((16,), jnp.float32)

  pltpu.sync_copy(zb_v, deg_sh.at[pl.ds(s * 640, 640)])
  pltpu.sync_copy(ei3_hbm.at[1, pl.ds(wid * DEG_NB, DEG_NB)], id2_v)
  plsc.subcore_barrier()

  def body(g, carry):
    hs = [pltpu.async_copy(ones_v, deg_sh.at[id2_v.at[g * 6 + b]], sem,
                           add=True) for b in range(6)]
    for h in hs:
      h.wait()
    return carry

  lax.fori_loop(0, DEG_NB // 6, body, 0)

  @pl.when(wid < DEG_XTRA)
  def _():
    pltpu.sync_copy(ei3_hbm.at[1, pl.ds(NW * DEG_NB + wid, 1)],
                    id2_v.at[pl.ds(0, 1)])
    pltpu.sync_copy(ones_v, deg_sh.at[id2_v.at[0]], add=True)

  plsc.subcore_barrier()

  pltpu.sync_copy(deg_sh.at[pl.ds(s * 640, 640)], zb_v)
  pltpu.sync_copy(zb_v, out_hbm.at[c, pl.ds(s * 640, 640)])


def _make_sc_prop_fs(d):
  dh = d // 2
  half = jax.ShapeDtypeStruct((N_PAD, dh), jnp.float32)

  @functools.partial(
      pl.kernel,
      out_type=[half, half],
      mesh=_MESH,
      compiler_params=_SC_PARAMS,
      scratch_types=[
          pltpu.VMEM((FS_PH, KB), jnp.int32),
          pltpu.VMEM((FS_PH, KB), jnp.int32),
          pltpu.VMEM((KB, dh), jnp.float32),
          pltpu.VMEM((KB, dh), jnp.float32),
          pltpu.VMEM_SHARED((N_PAD, dh), jnp.float32),
          pltpu.VMEM_SHARED((N_PAD, dh), jnp.float32),
          pltpu.SemaphoreType.DMA,
          pltpu.SemaphoreType.DMA,
      ],
  )
  def prop(g_hbm, ei3_hbm, out0_hbm, out1_hbm, is2_v, id2_v, rows0_v, rows1_v,
           g_sh, acc_sh, semg0, semg1):
    _, c, s = _worker_id()

    offs = list(range(0, dh - 15, 16))
    if offs[-1] != dh - 16:
      offs.append(dh - 16)
    def zinit(r, carry):
      for col in offs:
        rows0_v[r, pl.ds(col, 16)] = jnp.zeros((16,), jnp.float32)
      return carry
    lax.fori_loop(0, KB, zinit, 0)
    for j in range(5):
      pltpu.sync_copy(rows0_v, acc_sh.at[pl.ds(s * 640 + j * ZROWS, ZROWS)])
    for j in range(5):
      r0 = s * 640 + j * ZROWS
      pltpu.sync_copy(g_hbm.at[pl.ds(r0, ZROWS), pl.ds(c * dh, dh)], rows1_v)
      pltpu.sync_copy(rows1_v, g_sh.at[pl.ds(r0, ZROWS)])
    plsc.subcore_barrier()

    for ph in range(FS_NB // FS_PH):
      base = s * FS_NB + ph * FS_PH
      pltpu.sync_copy(ei3_hbm.at[0, pl.ds(base, FS_PH)], is2_v)
      pltpu.sync_copy(ei3_hbm.at[1, pl.ds(base, FS_PH)], id2_v)
      pltpu.async_copy(g_sh.at[is2_v.at[0]], rows0_v, semg0)

      def body(p, carry):
        j0 = p * 2
        pltpu.async_copy(g_sh.at[is2_v.at[j0 + 1]], rows1_v, semg1)
        pltpu.make_async_copy(g_sh.at[is2_v.at[j0]], rows0_v, semg0).wait()
        pltpu.sync_copy(rows0_v, acc_sh.at[id2_v.at[j0]], add=True)

        @pl.when(p < FS_PH // 2 - 1)
        def _():
          pltpu.async_copy(g_sh.at[is2_v.at[j0 + 2]], rows0_v, semg0)

        pltpu.make_async_copy(g_sh.at[is2_v.at[j0 + 1]], rows1_v,
                              semg1).wait()
        pltpu.sync_copy(rows1_v, acc_sh.at[id2_v.at[j0 + 1]], add=True)
        return carry

      lax.fori_loop(0, FS_PH // 2, body, 0)

    @pl.when(s < FS_XTRA)
    def _():
      blk = NS * FS_NB + s
      pltpu.sync_copy(ei3_hbm.at[0, pl.ds(blk, 1)], is2_v.at[pl.ds(0, 1)])
      pltpu.sync_copy(ei3_hbm.at[1, pl.ds(blk, 1)], id2_v.at[pl.ds(0, 1)])
      pltpu.async_copy(g_sh.at[is2_v.at[0]], rows0_v, semg0).wait()
      pltpu.sync_copy(rows0_v, acc_sh.at[id2_v.at[0]], add=True)

    plsc.subcore_barrier()

    for j in range(5):
      r0 = s * 640 + j * ZROWS
      pltpu.sync_copy(acc_sh.at[pl.ds(r0, ZROWS)], rows0_v)

      @pl.when(c == 0)
      def _():
        pltpu.sync_copy(rows0_v, out0_hbm.at[pl.ds(r0, ZROWS)])

      @pl.when(c == 1)
      def _():
        pltpu.sync_copy(rows0_v, out1_hbm.at[pl.ds(r0, ZROWS)])

  return prop


_sc_prop_fs_128 = _make_sc_prop_fs(128)
_sc_prop_fs_48 = _make_sc_prop_fs(48)


_RB = 1280


def _tc1a_body(x_ref, w1_ref, h1_ref):
  h1_ref[...] = lax.dot_general(x_ref[...], w1_ref[...],
                                (((1,), (1,)), ((), ())),
                                preferred_element_type=jnp.float32)


def _tc1a(x, w1):
  return pl.pallas_call(
      _tc1a_body,
      grid=(N_PAD // _RB,),
      in_specs=[
          pl.BlockSpec((_RB, 128), lambda i: (i, 0)),
          pl.BlockSpec((128, 128), lambda i: (0, 0)),
      ],
      out_specs=pl.BlockSpec((_RB, 128), lambda i: (i, 0)),
      out_shape=jax.ShapeDtypeStruct((N_PAD, 128), jnp.float32),
  )(x, w1)


def _tc1b_body(h1_ref, d0_ref, d1_ref, g1_ref, dinv_ref):
  deg = d0_ref[...] + d1_ref[...] + 1.0
  dinv = lax.rsqrt(jnp.maximum(deg, 1.0))
  g1_ref[...] = dinv * h1_ref[...]
  dinv_ref[...] = dinv


def _tc1b(h1, d0, d1):
  return pl.pallas_call(
      _tc1b_body,
      grid=(N_PAD // _RB,),
      in_specs=[
          pl.BlockSpec((_RB, 128), lambda i: (i, 0)),
          pl.BlockSpec((_RB, 1), lambda i: (i, 0)),
          pl.BlockSpec((_RB, 1), lambda i: (i, 0)),
      ],
      out_specs=[
          pl.BlockSpec((_RB, 128), lambda i: (i, 0)),
          pl.BlockSpec((_RB, 1), lambda i: (i, 0)),
      ],
      out_shape=[
          jax.ShapeDtypeStruct((N_PAD, 128), jnp.float32),
          jax.ShapeDtypeStruct((N_PAD, 1), jnp.float32),
      ],
  )(h1, d0, d1)


def _tc2_body(h0_ref, h1_ref, g1_ref, dinv_ref, b1_ref, w2_ref, g2_ref):
  dinv = dinv_ref[...]
  s1 = jnp.concatenate([h0_ref[...], h1_ref[...]], axis=1)
  t = (s1 + g1_ref[...]) * dinv + b1_ref[...]
  o1 = jnp.maximum(t, 0.0)
  h2 = lax.dot_general(o1, w2_ref[...], (((1,), (1,)), ((), ())),
                       preferred_element_type=jnp.float32)
  g2_ref[...] = dinv * h2


def _tc2(h0, h1, g1, dinv, b1, w2p):
  return pl.pallas_call(
      _tc2_body,
      grid=(N_PAD // _RB,),
      in_specs=[
          pl.BlockSpec((_RB, 64), lambda i: (i, 0)),
          pl.BlockSpec((_RB, 64), lambda i: (i, 0)),
          pl.BlockSpec((_RB, 128), lambda i: (i, 0)),
          pl.BlockSpec((_RB, 1), lambda i: (i, 0)),
          pl.BlockSpec((1, 128), lambda i: (0, 0)),
          pl.BlockSpec((48, 128), lambda i: (0, 0)),
      ],
      out_specs=pl.BlockSpec((_RB, 48), lambda i: (i, 0)),
      out_shape=jax.ShapeDtypeStruct((N_PAD, 48), jnp.float32),
  )(h0, h1, g1, dinv, b1, w2p)


def _tc3_body(h0_ref, h1_ref, g2_ref, dinv_ref, b2_ref, out_ref):
  s2 = jnp.concatenate([h0_ref[...], h1_ref[...]], axis=1)
  out_ref[...] = (s2 + g2_ref[...]) * dinv_ref[...] + b2_ref[...]


def _tc3(h0, h1, g2, dinv, b2p):
  return pl.pallas_call(
      _tc3_body,
      grid=(N_PAD // _RB,),
      in_specs=[
          pl.BlockSpec((_RB, 24), lambda i: (i, 0)),
          pl.BlockSpec((_RB, 24), lambda i: (i, 0)),
          pl.BlockSpec((_RB, 48), lambda i: (i, 0)),
          pl.BlockSpec((_RB, 1), lambda i: (i, 0)),
          pl.BlockSpec((1, 48), lambda i: (0, 0)),
      ],
      out_specs=pl.BlockSpec((_RB, 48), lambda i: (i, 0)),
      out_shape=jax.ShapeDtypeStruct((N_PAD, 48), jnp.float32),
  )(h0, h1, g2, dinv, b2p)


def kernel(x, edge_index, W1, b1, W2, b2):
  ei3 = edge_index.astype(jnp.int32).reshape(2, NBLK, KB)

  h1 = _tc1a(x, W1)
  degp = _sc_degree(ei3)
  g1, dinv = _tc1b(h1, degp[0][:, None], degp[1][:, None])

  s1h0, s1h1 = _sc_prop_fs_128(g1, ei3)

  w2p = jnp.zeros((48, 128), jnp.float32).at[:40].set(W2)
  b1r = b1.reshape(1, 128)
  b2p = jnp.zeros((1, 48), jnp.float32).at[0, :40].set(b2)

  g2 = _tc2(s1h0, s1h1, g1, dinv, b1r, w2p)
  s2h0, s2h1 = _sc_prop_fs_48(g2, ei3)
  out = _tc3(s2h0, s2h1, g2, dinv, b2p)
  return out[:N, :40]


# speedup vs baseline: 1.2448x; 1.0191x over previous
import functools

import jax
import jax.numpy as jnp
from jax import lax
from jax.experimental import pallas as pl
from jax.experimental.pallas import tpu as pltpu
from jax.experimental.pallas import tpu_sc as plsc

N = 10000
E = 320000
NC = 2
NS = 16
NW = NC * NS
KB = 128
NBLK = E // KB

N_PAD = 10240
ZROWS = 128

DEG_NB = NBLK // NW
DEG_XTRA = NBLK - DEG_NB * NW

FS_NB = NBLK // NS
FS_XTRA = NBLK - FS_NB * NS

_MESH = plsc.VectorSubcoreMesh(
    core_axis_name="c", subcore_axis_name="s", num_cores=NC, num_subcores=NS)

_SC_PARAMS = pltpu.CompilerParams(use_tc_tiling_on_sc=False)


def _worker_id():
  c = lax.axis_index("c")
  s = lax.axis_index("s")
  return c * NS + s, c, s


@functools.partial(
    pl.kernel,
    out_type=jax.ShapeDtypeStruct((NC, N_PAD), jnp.float32),
    mesh=_MESH,
    compiler_params=_SC_PARAMS,
    scratch_types=[
        pltpu.VMEM((DEG_NB, KB), jnp.int32),
        pltpu.VMEM((KB,), jnp.float32),
        pltpu.VMEM((640,), jnp.float32),
        pltpu.VMEM_SHARED((N_PAD,), jnp.float32),
        pltpu.SemaphoreType.DMA,
    ],
)
def _sc_degree(ei3_hbm, out_hbm, id2_v, ones_v, zb_v, deg_sh, sem):
  wid, c, s = _worker_id()

  for i in range(KB // 16):
    ones_v[pl.ds(i * 16, 16)] = jnp.full((16,), 1.0, jnp.float32)
  for i in range(640 // 16):
    zb_v[pl.ds(i * 16, 16)] = jnp.zeros((16,), jnp.float32)

  pltpu.sync_copy(zb_v, deg_sh.at[pl.ds(s * 640, 640)])
  pltpu.sync_copy(ei3_hbm.at[1, pl.ds(wid * DEG_NB, DEG_NB)], id2_v)
  plsc.subcore_barrier()

  def body(g, carry):
    hs = [pltpu.async_copy(ones_v, deg_sh.at[id2_v.at[g * 6 + b]], sem,
                           add=True) for b in range(6)]
    for h in hs:
      h.wait()
    return carry

  lax.fori_loop(0, DEG_NB // 6, body, 0)

  @pl.when(wid < DEG_XTRA)
  def _():
    pltpu.sync_copy(ei3_hbm.at[1, pl.ds(NW * DEG_NB + wid, 1)],
                    id2_v.at[pl.ds(0, 1)])
    pltpu.sync_copy(ones_v, deg_sh.at[id2_v.at[0]], add=True)

  plsc.subcore_barrier()

  pltpu.sync_copy(deg_sh.at[pl.ds(s * 640, 640)], zb_v)
  pltpu.sync_copy(zb_v, out_hbm.at[c, pl.ds(s * 640, 640)])


def _make_sc_prop_fs(d, fs_ph):
  dh = d // 2
  half = jax.ShapeDtypeStruct((N_PAD, dh), jnp.float32)

  @functools.partial(
      pl.kernel,
      out_type=[half, half],
      mesh=_MESH,
      compiler_params=_SC_PARAMS,
      scratch_types=[
          pltpu.VMEM((fs_ph, KB), jnp.int32),
          pltpu.VMEM((fs_ph, KB), jnp.int32),
          pltpu.VMEM((KB, dh), jnp.float32),
          pltpu.VMEM((KB, dh), jnp.float32),
          pltpu.VMEM_SHARED((N_PAD, dh), jnp.float32),
          pltpu.VMEM_SHARED((N_PAD, dh), jnp.float32),
          pltpu.SemaphoreType.DMA,
          pltpu.SemaphoreType.DMA,
      ],
  )
  def prop(g_hbm, ei3_hbm, out0_hbm, out1_hbm, is2_v, id2_v, rows0_v, rows1_v,
           g_sh, acc_sh, semg0, semg1):
    _, c, s = _worker_id()

    offs = list(range(0, dh - 15, 16))
    if offs[-1] != dh - 16:
      offs.append(dh - 16)
    def zinit(r, carry):
      for col in offs:
        rows0_v[r, pl.ds(col, 16)] = jnp.zeros((16,), jnp.float32)
      return carry
    lax.fori_loop(0, KB, zinit, 0)
    for j in range(5):
      pltpu.sync_copy(rows0_v, acc_sh.at[pl.ds(s * 640 + j * ZROWS, ZROWS)])
    for j in range(5):
      r0 = s * 640 + j * ZROWS
      pltpu.sync_copy(g_hbm.at[pl.ds(r0, ZROWS), pl.ds(c * dh, dh)], rows1_v)
      pltpu.sync_copy(rows1_v, g_sh.at[pl.ds(r0, ZROWS)])
    plsc.subcore_barrier()

    for ph in range(FS_NB // fs_ph):
      base = s * FS_NB + ph * fs_ph
      pltpu.sync_copy(ei3_hbm.at[0, pl.ds(base, fs_ph)], is2_v)
      pltpu.sync_copy(ei3_hbm.at[1, pl.ds(base, fs_ph)], id2_v)
      pltpu.async_copy(g_sh.at[is2_v.at[0]], rows0_v, semg0)

      def body(p, carry):
        j0 = p * 2
        pltpu.async_copy(g_sh.at[is2_v.at[j0 + 1]], rows1_v, semg1)
        pltpu.make_async_copy(g_sh.at[is2_v.at[j0]], rows0_v, semg0).wait()
        pltpu.sync_copy(rows0_v, acc_sh.at[id2_v.at[j0]], add=True)

        @pl.when(p < fs_ph // 2 - 1)
        def _():
          pltpu.async_copy(g_sh.at[is2_v.at[j0 + 2]], rows0_v, semg0)

        pltpu.make_async_copy(g_sh.at[is2_v.at[j0 + 1]], rows1_v,
                              semg1).wait()
        pltpu.sync_copy(rows1_v, acc_sh.at[id2_v.at[j0 + 1]], add=True)
        return carry

      lax.fori_loop(0, fs_ph // 2, body, 0)

    @pl.when(s < FS_XTRA)
    def _():
      blk = NS * FS_NB + s
      pltpu.sync_copy(ei3_hbm.at[0, pl.ds(blk, 1)], is2_v.at[pl.ds(0, 1)])
      pltpu.sync_copy(ei3_hbm.at[1, pl.ds(blk, 1)], id2_v.at[pl.ds(0, 1)])
      pltpu.async_copy(g_sh.at[is2_v.at[0]], rows0_v, semg0).wait()
      pltpu.sync_copy(rows0_v, acc_sh.at[id2_v.at[0]], add=True)

    plsc.subcore_barrier()

    for j in range(5):
      r0 = s * 640 + j * ZROWS
      pltpu.sync_copy(acc_sh.at[pl.ds(r0, ZROWS)], rows0_v)

      @pl.when(c == 0)
      def _():
        pltpu.sync_copy(rows0_v, out0_hbm.at[pl.ds(r0, ZROWS)])

      @pl.when(c == 1)
      def _():
        pltpu.sync_copy(rows0_v, out1_hbm.at[pl.ds(r0, ZROWS)])

  return prop


_sc_prop_fs_128 = _make_sc_prop_fs(128, 78)
_sc_prop_fs_48 = _make_sc_prop_fs(48, 156)


_RB = 1280


def _tc1a_body(x_ref, w1_ref, h1_ref):
  h1_ref[...] = lax.dot_general(x_ref[...], w1_ref[...],
                                (((1,), (1,)), ((), ())),
                                preferred_element_type=jnp.float32)


def _tc1a(x, w1):
  return pl.pallas_call(
      _tc1a_body,
      grid=(N_PAD // _RB,),
      in_specs=[
          pl.BlockSpec((_RB, 128), lambda i: (i, 0)),
          pl.BlockSpec((128, 128), lambda i: (0, 0)),
      ],
      out_specs=pl.BlockSpec((_RB, 128), lambda i: (i, 0)),
      out_shape=jax.ShapeDtypeStruct((N_PAD, 128), jnp.float32),
  )(x, w1)


def _tc1b_body(h1_ref, d0_ref, d1_ref, g1_ref, dinv_ref):
  deg = d0_ref[...] + d1_ref[...] + 1.0
  dinv = lax.rsqrt(jnp.maximum(deg, 1.0))
  g1_ref[...] = dinv * h1_ref[...]
  dinv_ref[...] = dinv


def _tc1b(h1, d0, d1):
  return pl.pallas_call(
      _tc1b_body,
      grid=(N_PAD // _RB,),
      in_specs=[
          pl.BlockSpec((_RB, 128), lambda i: (i, 0)),
          pl.BlockSpec((_RB, 1), lambda i: (i, 0)),
          pl.BlockSpec((_RB, 1), lambda i: (i, 0)),
      ],
      out_specs=[
          pl.BlockSpec((_RB, 128), lambda i: (i, 0)),
          pl.BlockSpec((_RB, 1), lambda i: (i, 0)),
      ],
      out_shape=[
          jax.ShapeDtypeStruct((N_PAD, 128), jnp.float32),
          jax.ShapeDtypeStruct((N_PAD, 1), jnp.float32),
      ],
  )(h1, d0, d1)


def _tc2_body(h0_ref, h1_ref, g1_ref, dinv_ref, b1_ref, w2_ref, g2_ref):
  dinv = dinv_ref[...]
  s1 = jnp.concatenate([h0_ref[...], h1_ref[...]], axis=1)
  t = (s1 + g1_ref[...]) * dinv + b1_ref[...]
  o1 = jnp.maximum(t, 0.0)
  h2 = lax.dot_general(o1, w2_ref[...], (((1,), (1,)), ((), ())),
                       preferred_element_type=jnp.float32)
  g2_ref[...] = dinv * h2


def _tc2(h0, h1, g1, dinv, b1, w2p):
  return pl.pallas_call(
      _tc2_body,
      grid=(N_PAD // _RB,),
      in_specs=[
          pl.BlockSpec((_RB, 64), lambda i: (i, 0)),
          pl.BlockSpec((_RB, 64), lambda i: (i, 0)),
          pl.BlockSpec((_RB, 128), lambda i: (i, 0)),
          pl.BlockSpec((_RB, 1), lambda i: (i, 0)),
          pl.BlockSpec((1, 128), lambda i: (0, 0)),
          pl.BlockSpec((48, 128), lambda i: (0, 0)),
      ],
      out_specs=pl.BlockSpec((_RB, 48), lambda i: (i, 0)),
      out_shape=jax.ShapeDtypeStruct((N_PAD, 48), jnp.float32),
  )(h0, h1, g1, dinv, b1, w2p)


def _tc3_body(h0_ref, h1_ref, g2_ref, dinv_ref, b2_ref, out_ref):
  s2 = jnp.concatenate([h0_ref[...], h1_ref[...]], axis=1)
  out_ref[...] = (s2 + g2_ref[...]) * dinv_ref[...] + b2_ref[...]


def _tc3(h0, h1, g2, dinv, b2p):
  return pl.pallas_call(
      _tc3_body,
      grid=(N_PAD // _RB,),
      in_specs=[
          pl.BlockSpec((_RB, 24), lambda i: (i, 0)),
          pl.BlockSpec((_RB, 24), lambda i: (i, 0)),
          pl.BlockSpec((_RB, 48), lambda i: (i, 0)),
          pl.BlockSpec((_RB, 1), lambda i: (i, 0)),
          pl.BlockSpec((1, 48), lambda i: (0, 0)),
      ],
      out_specs=pl.BlockSpec((_RB, 48), lambda i: (i, 0)),
      out_shape=jax.ShapeDtypeStruct((N_PAD, 48), jnp.float32),
  )(h0, h1, g2, dinv, b2p)


def kernel(x, edge_index, W1, b1, W2, b2):
  ei3 = edge_index.astype(jnp.int32).reshape(2, NBLK, KB)

  h1 = _tc1a(x, W1)
  degp = _sc_degree(ei3)
  g1, dinv = _tc1b(h1, degp[0][:, None], degp[1][:, None])

  s1h0, s1h1 = _sc_prop_fs_128(g1, ei3)

  w2p = jnp.zeros((48, 128), jnp.float32).at[:40].set(W2)
  b1r = b1.reshape(1, 128)
  b2p = jnp.zeros((1, 48), jnp.float32).at[0, :40].set(b2)

  g2 = _tc2(s1h0, s1h1, g1, dinv, b1r, w2p)
  s2h0, s2h1 = _sc_prop_fs_48(g2, ei3)
  out = _tc3(s2h0, s2h1, g2, dinv, b2p)
  return out[:N, :40]


# speedup vs baseline: 1.2489x; 1.0033x over previous
import functools

import jax
import jax.numpy as jnp
from jax import lax
from jax.experimental import pallas as pl
from jax.experimental.pallas import tpu as pltpu
from jax.experimental.pallas import tpu_sc as plsc

N = 10000
E = 320000
NC = 2
NS = 16
NW = NC * NS
KB = 128
NBLK = E // KB

N_PAD = 10240
ZROWS = 128

DEG_NB = NBLK // NW
DEG_XTRA = NBLK - DEG_NB * NW

FS_NB = NBLK // NS
FS_XTRA = NBLK - FS_NB * NS

_MESH = plsc.VectorSubcoreMesh(
    core_axis_name="c", subcore_axis_name="s", num_cores=NC, num_subcores=NS)

_SC_PARAMS = pltpu.CompilerParams(use_tc_tiling_on_sc=False)


def _worker_id():
  c = lax.axis_index("c")
  s = lax.axis_index("s")
  return c * NS + s, c, s


@functools.partial(
    pl.kernel,
    out_type=jax.ShapeDtypeStruct((NC, N_PAD), jnp.float32),
    mesh=_MESH,
    compiler_params=_SC_PARAMS,
    scratch_types=[
        pltpu.VMEM((DEG_NB, KB), jnp.int32),
        pltpu.VMEM((KB,), jnp.float32),
        pltpu.VMEM((640,), jnp.float32),
        pltpu.VMEM_SHARED((N_PAD,), jnp.float32),
        pltpu.SemaphoreType.DMA,
    ],
)
def _sc_degree(ei3_hbm, out_hbm, id2_v, ones_v, zb_v, deg_sh, sem):
  wid, c, s = _worker_id()

  for i in range(KB // 16):
    ones_v[pl.ds(i * 16, 16)] = jnp.full((16,), 1.0, jnp.float32)
  for i in range(640 // 16):
    zb_v[pl.ds(i * 16, 16)] = jnp.zeros((16,), jnp.float32)

  pltpu.sync_copy(zb_v, deg_sh.at[pl.ds(s * 640, 640)])
  pltpu.sync_copy(ei3_hbm.at[1, pl.ds(wid * DEG_NB, DEG_NB)], id2_v)
  plsc.subcore_barrier()

  def body(g, carry):
    hs = [pltpu.async_copy(ones_v, deg_sh.at[id2_v.at[g * 6 + b]], sem,
                           add=True) for b in range(6)]
    for h in hs:
      h.wait()
    return carry

  lax.fori_loop(0, DEG_NB // 6, body, 0)

  @pl.when(wid < DEG_XTRA)
  def _():
    pltpu.sync_copy(ei3_hbm.at[1, pl.ds(NW * DEG_NB + wid, 1)],
                    id2_v.at[pl.ds(0, 1)])
    pltpu.sync_copy(ones_v, deg_sh.at[id2_v.at[0]], add=True)

  plsc.subcore_barrier()

  pltpu.sync_copy(deg_sh.at[pl.ds(s * 640, 640)], zb_v)
  pltpu.sync_copy(zb_v, out_hbm.at[c, pl.ds(s * 640, 640)])


def _make_sc_prop_fs(d, fs_ph):
  dh = d // 2
  half = jax.ShapeDtypeStruct((N_PAD, dh), jnp.float32)

  @functools.partial(
      pl.kernel,
      out_type=[half, half],
      mesh=_MESH,
      compiler_params=_SC_PARAMS,
      scratch_types=[
          pltpu.VMEM((fs_ph, KB), jnp.int32),
          pltpu.VMEM((fs_ph, KB), jnp.int32),
          pltpu.VMEM((KB, dh), jnp.float32),
          pltpu.VMEM((KB, dh), jnp.float32),
          pltpu.VMEM_SHARED((N_PAD, dh), jnp.float32),
          pltpu.VMEM_SHARED((N_PAD, dh), jnp.float32),
          pltpu.SemaphoreType.DMA,
          pltpu.SemaphoreType.DMA,
      ],
  )
  def prop(g_hbm, ei3_hbm, out0_hbm, out1_hbm, is2_v, id2_v, rows0_v, rows1_v,
           g_sh, acc_sh, semg0, semg1):
    _, c, s = _worker_id()

    offs = list(range(0, dh - 15, 16))
    if offs[-1] != dh - 16:
      offs.append(dh - 16)
    def zinit(r, carry):
      for col in offs:
        rows0_v[r, pl.ds(col, 16)] = jnp.zeros((16,), jnp.float32)
      return carry
    lax.fori_loop(0, KB, zinit, 0)
    for j in range(5):
      pltpu.sync_copy(rows0_v, acc_sh.at[pl.ds(s * 640 + j * ZROWS, ZROWS)])
    for j in range(5):
      r0 = s * 640 + j * ZROWS
      pltpu.sync_copy(g_hbm.at[pl.ds(r0, ZROWS), pl.ds(c * dh, dh)], rows1_v)
      pltpu.sync_copy(rows1_v, g_sh.at[pl.ds(r0, ZROWS)])
    plsc.subcore_barrier()

    for ph in range(FS_NB // fs_ph):
      base = s * FS_NB + ph * fs_ph
      pltpu.sync_copy(ei3_hbm.at[0, pl.ds(base, fs_ph)], is2_v)
      pltpu.sync_copy(ei3_hbm.at[1, pl.ds(base, fs_ph)], id2_v)
      pltpu.async_copy(g_sh.at[is2_v.at[0]], rows0_v, semg0)

      def body(p, carry):
        j0 = p * 2
        pltpu.async_copy(g_sh.at[is2_v.at[j0 + 1]], rows1_v, semg1)
        pltpu.make_async_copy(g_sh.at[is2_v.at[j0]], rows0_v, semg0).wait()
        pltpu.sync_copy(rows0_v, acc_sh.at[id2_v.at[j0]], add=True)

        @pl.when(p < fs_ph // 2 - 1)
        def _():
          pltpu.async_copy(g_sh.at[is2_v.at[j0 + 2]], rows0_v, semg0)

        pltpu.make_async_copy(g_sh.at[is2_v.at[j0 + 1]], rows1_v,
                              semg1).wait()
        pltpu.sync_copy(rows1_v, acc_sh.at[id2_v.at[j0 + 1]], add=True)
        return carry

      lax.fori_loop(0, fs_ph // 2, body, 0)

    @pl.when(s < FS_XTRA)
    def _():
      blk = NS * FS_NB + s
      pltpu.sync_copy(ei3_hbm.at[0, pl.ds(blk, 1)], is2_v.at[pl.ds(0, 1)])
      pltpu.sync_copy(ei3_hbm.at[1, pl.ds(blk, 1)], id2_v.at[pl.ds(0, 1)])
      pltpu.async_copy(g_sh.at[is2_v.at[0]], rows0_v, semg0).wait()
      pltpu.sync_copy(rows0_v, acc_sh.at[id2_v.at[0]], add=True)

    plsc.subcore_barrier()

    for j in range(5):
      r0 = s * 640 + j * ZROWS
      pltpu.sync_copy(acc_sh.at[pl.ds(r0, ZROWS)], rows0_v)

      @pl.when(c == 0)
      def _():
        pltpu.sync_copy(rows0_v, out0_hbm.at[pl.ds(r0, ZROWS)])

      @pl.when(c == 1)
      def _():
        pltpu.sync_copy(rows0_v, out1_hbm.at[pl.ds(r0, ZROWS)])

  return prop


_sc_prop_fs_128 = _make_sc_prop_fs(128, 78)
_sc_prop_fs_48 = _make_sc_prop_fs(48, 156)


_RB = 1280


def _tc1a_body(x_ref, w1_ref, h1_ref):
  h1_ref[...] = lax.dot_general(x_ref[...], w1_ref[...],
                                (((1,), (1,)), ((), ())),
                                preferred_element_type=jnp.float32)


def _tc1a(x, w1):
  return pl.pallas_call(
      _tc1a_body,
      grid=(N_PAD // _RB,),
      in_specs=[
          pl.BlockSpec((_RB, 128), lambda i: (i, 0)),
          pl.BlockSpec((128, 128), lambda i: (0, 0)),
      ],
      out_specs=pl.BlockSpec((_RB, 128), lambda i: (i, 0)),
      out_shape=jax.ShapeDtypeStruct((N_PAD, 128), jnp.float32),
  )(x, w1)


def _tc1b_body(h1_ref, d0_ref, d1_ref, g1_ref, dinv_ref):
  deg = d0_ref[...] + d1_ref[...] + 1.0
  dinv = lax.rsqrt(jnp.maximum(deg, 1.0))
  g1_ref[...] = dinv * h1_ref[...]
  dinv_ref[...] = dinv


def _tc1b(h1, d0, d1):
  return pl.pallas_call(
      _tc1b_body,
      grid=(N_PAD // _RB,),
      in_specs=[
          pl.BlockSpec((_RB, 128), lambda i: (i, 0)),
          pl.BlockSpec((_RB, 1), lambda i: (i, 0)),
          pl.BlockSpec((_RB, 1), lambda i: (i, 0)),
      ],
      out_specs=[
          pl.BlockSpec((_RB, 128), lambda i: (i, 0)),
          pl.BlockSpec((_RB, 1), lambda i: (i, 0)),
      ],
      out_shape=[
          jax.ShapeDtypeStruct((N_PAD, 128), jnp.float32),
          jax.ShapeDtypeStruct((N_PAD, 1), jnp.float32),
      ],
  )(h1, d0, d1)


def _tc2_body(h0_ref, h1_ref, g1_ref, dinv_ref, b1_ref, w2_ref, g2_ref):
  dinv = dinv_ref[...]
  s1 = jnp.concatenate([h0_ref[...], h1_ref[...]], axis=1)
  t = (s1 + g1_ref[...]) * dinv + b1_ref[...]
  o1 = jnp.maximum(t, 0.0)
  h2 = lax.dot_general(o1, w2_ref[...], (((1,), (1,)), ((), ())),
                       preferred_element_type=jnp.float32)
  g2_ref[...] = dinv * h2


def _tc2(h0, h1, g1, dinv, b1, w2p):
  return pl.pallas_call(
      _tc2_body,
      grid=(N_PAD // _RB,),
      in_specs=[
          pl.BlockSpec((_RB, 64), lambda i: (i, 0)),
          pl.BlockSpec((_RB, 64), lambda i: (i, 0)),
          pl.BlockSpec((_RB, 128), lambda i: (i, 0)),
          pl.BlockSpec((_RB, 1), lambda i: (i, 0)),
          pl.BlockSpec((1, 128), lambda i: (0, 0)),
          pl.BlockSpec((48, 128), lambda i: (0, 0)),
      ],
      out_specs=pl.BlockSpec((_RB, 48), lambda i: (i, 0)),
      out_shape=jax.ShapeDtypeStruct((N_PAD, 48), jnp.float32),
  )(h0, h1, g1, dinv, b1, w2p)


def _tc3_body(h0_ref, h1_ref, g2_ref, dinv_ref, b2_ref, out_ref):
  s2 = jnp.concatenate([h0_ref[...], h1_ref[...]], axis=1)
  res = (s2 + g2_ref[...]) * dinv_ref[...] + b2_ref[...]
  out_ref[...] = res[:, :40]


def _tc3(h0, h1, g2, dinv, b2p):
  return pl.pallas_call(
      _tc3_body,
      grid=(N_PAD // _RB,),
      in_specs=[
          pl.BlockSpec((_RB, 24), lambda i: (i, 0)),
          pl.BlockSpec((_RB, 24), lambda i: (i, 0)),
          pl.BlockSpec((_RB, 48), lambda i: (i, 0)),
          pl.BlockSpec((_RB, 1), lambda i: (i, 0)),
          pl.BlockSpec((1, 48), lambda i: (0, 0)),
      ],
      out_specs=pl.BlockSpec((_RB, 40), lambda i: (i, 0)),
      out_shape=jax.ShapeDtypeStruct((N, 40), jnp.float32),
  )(h0, h1, g2, dinv, b2p)


def kernel(x, edge_index, W1, b1, W2, b2):
  ei3 = edge_index.astype(jnp.int32).reshape(2, NBLK, KB)

  h1 = _tc1a(x, W1)
  degp = _sc_degree(ei3)
  g1, dinv = _tc1b(h1, degp[0][:, None], degp[1][:, None])

  s1h0, s1h1 = _sc_prop_fs_128(g1, ei3)

  w2p = jnp.zeros((48, 128), jnp.float32).at[:40].set(W2)
  b1r = b1.reshape(1, 128)
  b2p = jnp.zeros((1, 48), jnp.float32).at[0, :40].set(b2)

  g2 = _tc2(s1h0, s1h1, g1, dinv, b1r, w2p)
  s2h0, s2h1 = _sc_prop_fs_48(g2, ei3)
  return _tc3(s2h0, s2h1, g2, dinv, b2p)
